# baseline (speedup 1.0000x reference)
import functools

import jax
import jax.numpy as jnp
from jax.experimental import pallas as pl
from jax.experimental.pallas import tpu as pltpu

_TOPK = 64


def _topk_kernel(x_ref, h_ref, f_ref):
    x = x_ref[0]
    h = h_ref[...]
    h64 = h[:, :64]
    hn2 = x.shape[0]
    int_max = jnp.int32(2147483647)

    def body(k, carry):
        mp, ipr = carry
        s = x - h
        kk = jnp.abs(s)
        row_i = jax.lax.broadcasted_iota(jnp.int32, (hn2, 128), 0)
        lane_i = jax.lax.broadcasted_iota(jnp.int32, (hn2, 128), 1)
        sid = 4 * row_i + jnp.where(lane_i >= 64, 2, 0) + jnp.where(s < 0, 1, 0)
        alive = (kk < mp) | ((kk == mp) & (sid > ipr))
        m = jnp.max(jnp.where(alive, kk, -1.0), axis=0, keepdims=True)
        m64 = jnp.maximum(m[:, :64], m[:, 64:])
        mt = jnp.concatenate([m64, m64], axis=1)
        sel = alive & (kk == mt)
        i = jnp.min(jnp.where(sel, sid, int_max), axis=0, keepdims=True)
        i64 = jnp.minimum(i[:, :64], i[:, 64:])
        v64 = h64 + jnp.where((i64 & 1) == 1, -m64, m64)
        f_ref[0, pl.ds(k, 1), :] = v64
        return (mt, jnp.concatenate([i64, i64], axis=1))

    mp0 = jnp.full((1, 128), jnp.inf, jnp.float32)
    ip0 = jnp.full((1, 128), -1, jnp.int32)
    jax.lax.fori_loop(0, _TOPK, body, (mp0, ip0))


def _head_kernel(flat_ref, mlp_WT_ref, mlp_b_ref, avg_ref,
                 Wr_WT_ref, Wr_b_ref, Wz_WT_ref, Wz_b_ref,
                 Wt_WT_ref, Wt_b_ref, out_ref):
    flat = flat_ref[0]
    tsi = jnp.dot(flat, mlp_WT_ref[...],
                  preferred_element_type=jnp.float32) + mlp_b_ref[...]
    avg = avg_ref[0]
    comb = jnp.concatenate([tsi, avg], axis=1)
    r = jax.nn.sigmoid(jnp.dot(comb, Wr_WT_ref[...],
                               preferred_element_type=jnp.float32) + Wr_b_ref[...])
    z = jax.nn.sigmoid(jnp.dot(comb, Wz_WT_ref[...],
                               preferred_element_type=jnp.float32) + Wz_b_ref[...])
    comb2 = jnp.concatenate([tsi, avg * r], axis=1)
    ht = jnp.tanh(jnp.dot(comb2, Wt_WT_ref[...],
                          preferred_element_type=jnp.float32) + Wt_b_ref[...])
    hn = z * ht + (1.0 - z) * tsi
    rows = jax.lax.broadcasted_iota(jnp.int32, (64, 64), 0)
    cols = jax.lax.broadcasted_iota(jnp.int32, (64, 64), 1)
    eye = (rows == cols).astype(jnp.float32)
    hcol = jax.lax.dot_general(eye, hn, (((1,), (1,)), ((), ())),
                               preferred_element_type=jnp.float32)
    out_ref[0] = jnp.broadcast_to(hcol, out_ref.shape[1:])


@functools.partial(jax.jit, static_argnames=())
def kernel(tem_emb, init_hidden, mlp_W, mlp_b, Wr_W, Wr_b, Wz_W, Wz_b,
           Wt_W, Wt_b, year):
    b, n, d = tem_emb.shape
    k = _TOPK
    hn2 = n // 2
    x2 = tem_emb.reshape(b, hn2, 2 * d)
    hrow = init_hidden.reshape(1, d)
    h2 = jnp.concatenate([hrow, hrow], axis=1)

    feats = pl.pallas_call(
        _topk_kernel,
        grid=(b,),
        in_specs=[
            pl.BlockSpec((1, hn2, 2 * d), lambda i: (i, 0, 0)),
            pl.BlockSpec((1, 2 * d), lambda i: (0, 0)),
        ],
        out_specs=pl.BlockSpec((1, k, d), lambda i: (i, 0, 0)),
        out_shape=jax.ShapeDtypeStruct((b, k, d), jnp.float32),
    )(x2, h2)

    flat = jnp.transpose(feats, (0, 2, 1)).reshape(b, 1, d * k)
    avg = jnp.broadcast_to(hrow[None], (b, 1, d))

    out = pl.pallas_call(
        _head_kernel,
        grid=(b,),
        in_specs=[
            pl.BlockSpec((1, 1, d * k), lambda i: (i, 0, 0)),
            pl.BlockSpec((d * k, 64), lambda i: (0, 0)),
            pl.BlockSpec((1, 64), lambda i: (0, 0)),
            pl.BlockSpec((1, 1, d), lambda i: (i, 0, 0)),
            pl.BlockSpec((2 * 64, 64), lambda i: (0, 0)),
            pl.BlockSpec((1, 64), lambda i: (0, 0)),
            pl.BlockSpec((2 * 64, 64), lambda i: (0, 0)),
            pl.BlockSpec((1, 64), lambda i: (0, 0)),
            pl.BlockSpec((2 * 64, 64), lambda i: (0, 0)),
            pl.BlockSpec((1, 64), lambda i: (0, 0)),
        ],
        out_specs=pl.BlockSpec((1, 64, n), lambda i: (i, 0, 0)),
        out_shape=jax.ShapeDtypeStruct((b, 64, n), jnp.float32),
    )(flat, mlp_W.T, mlp_b.reshape(1, 64), avg,
      Wr_W.T, Wr_b.reshape(1, 64), Wz_W.T, Wz_b.reshape(1, 64),
      Wt_W.T, Wt_b.reshape(1, 64))
    return out


# speedup vs baseline: 1.0969x; 1.0969x over previous
import functools

import jax
import jax.numpy as jnp
from jax.experimental import pallas as pl
from jax.experimental.pallas import tpu as pltpu

_TOPK = 64


def _topk_kernel(x_ref, h_ref, f_ref):
    x = x_ref[0]
    h = h_ref[...]
    h64 = h[:, :64]
    hn2 = x.shape[0]
    int_max = jnp.int32(2147483647)

    lane_i = jax.lax.broadcasted_iota(jnp.int32, (1, 128), 1)
    lane2 = jnp.where(lane_i >= 64, 2, 0)

    def body(k, carry):
        mp, ipr = carry
        s = x - h
        kk = jnp.abs(s)
        row_i = jax.lax.broadcasted_iota(jnp.int32, (hn2, 128), 0)
        sgn = jax.lax.shift_right_logical(
            jax.lax.bitcast_convert_type(s, jnp.int32), 31)
        sid = (jax.lax.shift_left(row_i, 2) | lane2) | sgn
        alive = (kk < mp) | ((kk == mp) & (sid > ipr))
        m = jnp.max(jnp.where(alive, kk, -1.0), axis=0, keepdims=True)
        m64 = jnp.maximum(m[:, :64], m[:, 64:])
        mt = jnp.concatenate([m64, m64], axis=1)
        sel = alive & (kk == mt)
        i = jnp.min(jnp.where(sel, sid, int_max), axis=0, keepdims=True)
        i64 = jnp.minimum(i[:, :64], i[:, 64:])
        v64 = h64 + jnp.where((i64 & 1) == 1, -m64, m64)
        f_ref[0, pl.ds(k, 1), :] = v64
        return (mt, jnp.concatenate([i64, i64], axis=1))

    mp0 = jnp.full((1, 128), jnp.inf, jnp.float32)
    ip0 = jnp.full((1, 128), -1, jnp.int32)
    jax.lax.fori_loop(0, _TOPK, body, (mp0, ip0))


def _head_kernel(flat_ref, mlp_WT_ref, mlp_b_ref, avg_ref,
                 Wr_WT_ref, Wr_b_ref, Wz_WT_ref, Wz_b_ref,
                 Wt_WT_ref, Wt_b_ref, out_ref):
    flat = flat_ref[0]
    tsi = jnp.dot(flat, mlp_WT_ref[...],
                  preferred_element_type=jnp.float32) + mlp_b_ref[...]
    avg = avg_ref[0]
    comb = jnp.concatenate([tsi, avg], axis=1)
    r = jax.nn.sigmoid(jnp.dot(comb, Wr_WT_ref[...],
                               preferred_element_type=jnp.float32) + Wr_b_ref[...])
    z = jax.nn.sigmoid(jnp.dot(comb, Wz_WT_ref[...],
                               preferred_element_type=jnp.float32) + Wz_b_ref[...])
    comb2 = jnp.concatenate([tsi, avg * r], axis=1)
    ht = jnp.tanh(jnp.dot(comb2, Wt_WT_ref[...],
                          preferred_element_type=jnp.float32) + Wt_b_ref[...])
    hn = z * ht + (1.0 - z) * tsi
    rows = jax.lax.broadcasted_iota(jnp.int32, (64, 64), 0)
    cols = jax.lax.broadcasted_iota(jnp.int32, (64, 64), 1)
    eye = (rows == cols).astype(jnp.float32)
    hcol = jax.lax.dot_general(eye, hn, (((1,), (1,)), ((), ())),
                               preferred_element_type=jnp.float32)
    out_ref[0] = jnp.broadcast_to(hcol, out_ref.shape[1:])


@functools.partial(jax.jit, static_argnames=())
def kernel(tem_emb, init_hidden, mlp_W, mlp_b, Wr_W, Wr_b, Wz_W, Wz_b,
           Wt_W, Wt_b, year):
    b, n, d = tem_emb.shape
    k = _TOPK
    hn2 = n // 2
    x2 = tem_emb.reshape(b, hn2, 2 * d)
    hrow = init_hidden.reshape(1, d)
    h2 = jnp.concatenate([hrow, hrow], axis=1)

    feats = pl.pallas_call(
        _topk_kernel,
        grid=(b,),
        in_specs=[
            pl.BlockSpec((1, hn2, 2 * d), lambda i: (i, 0, 0)),
            pl.BlockSpec((1, 2 * d), lambda i: (0, 0)),
        ],
        out_specs=pl.BlockSpec((1, k, d), lambda i: (i, 0, 0)),
        out_shape=jax.ShapeDtypeStruct((b, k, d), jnp.float32),
    )(x2, h2)

    flat = jnp.transpose(feats, (0, 2, 1)).reshape(b, 1, d * k)
    avg = jnp.broadcast_to(hrow[None], (b, 1, d))

    out = pl.pallas_call(
        _head_kernel,
        grid=(b,),
        in_specs=[
            pl.BlockSpec((1, 1, d * k), lambda i: (i, 0, 0)),
            pl.BlockSpec((d * k, 64), lambda i: (0, 0)),
            pl.BlockSpec((1, 64), lambda i: (0, 0)),
            pl.BlockSpec((1, 1, d), lambda i: (i, 0, 0)),
            pl.BlockSpec((2 * 64, 64), lambda i: (0, 0)),
            pl.BlockSpec((1, 64), lambda i: (0, 0)),
            pl.BlockSpec((2 * 64, 64), lambda i: (0, 0)),
            pl.BlockSpec((1, 64), lambda i: (0, 0)),
            pl.BlockSpec((2 * 64, 64), lambda i: (0, 0)),
            pl.BlockSpec((1, 64), lambda i: (0, 0)),
        ],
        out_specs=pl.BlockSpec((1, 64, n), lambda i: (i, 0, 0)),
        out_shape=jax.ShapeDtypeStruct((b, 64, n), jnp.float32),
    )(flat, mlp_W.T, mlp_b.reshape(1, 64), avg,
      Wr_W.T, Wr_b.reshape(1, 64), Wz_W.T, Wz_b.reshape(1, 64),
      Wt_W.T, Wt_b.reshape(1, 64))
    return out


# speedup vs baseline: 1.1499x; 1.0483x over previous
import functools

import jax
import jax.numpy as jnp
from jax.experimental import pallas as pl
from jax.experimental.pallas import tpu as pltpu

_TOPK = 64


def _topk_kernel(x_ref, h_ref, f_ref):
    x = x_ref[0]
    h = h_ref[...]
    h64 = h[:, :64]
    hn2 = x.shape[0]
    int_max = jnp.int32(2147483647)

    lane_i = jax.lax.broadcasted_iota(jnp.int32, (1, 128), 1)
    lane2 = jnp.where(lane_i >= 64, 2, 0)

    def body(k, carry):
        mp, ipr = carry
        s = x - h
        kk = jnp.abs(s)
        row_i = jax.lax.broadcasted_iota(jnp.int32, (hn2, 128), 0)
        sgn = jax.lax.shift_right_logical(
            jax.lax.bitcast_convert_type(s, jnp.int32), 31)
        sid = (jax.lax.shift_left(row_i, 2) | lane2) | sgn
        alive = (kk < mp) | ((kk == mp) & (sid > ipr))
        m = jnp.max(jnp.where(alive, kk, -1.0), axis=0, keepdims=True)
        m64 = jnp.maximum(m[:, :64], m[:, 64:])
        mt = jnp.concatenate([m64, m64], axis=1)
        thr64 = jnp.where(m64 == mp[:, :64], ipr[:, :64], -1)
        thr = jnp.concatenate([thr64, thr64], axis=1)
        sel = (kk == mt) & (sid > thr)
        i = jnp.min(jnp.where(sel, sid, int_max), axis=0, keepdims=True)
        i64 = jnp.minimum(i[:, :64], i[:, 64:])
        v64 = h64 + jnp.where((i64 & 1) == 1, -m64, m64)
        f_ref[0, pl.ds(k, 1), :] = v64
        return (mt, jnp.concatenate([i64, i64], axis=1))

    mp0 = jnp.full((1, 128), jnp.inf, jnp.float32)
    ip0 = jnp.full((1, 128), -1, jnp.int32)
    jax.lax.fori_loop(0, _TOPK, body, (mp0, ip0))


def _head_kernel(flat_ref, mlp_WT_ref, mlp_b_ref, avg_ref,
                 Wr_WT_ref, Wr_b_ref, Wz_WT_ref, Wz_b_ref,
                 Wt_WT_ref, Wt_b_ref, out_ref):
    flat = flat_ref[0]
    tsi = jnp.dot(flat, mlp_WT_ref[...],
                  preferred_element_type=jnp.float32) + mlp_b_ref[...]
    avg = avg_ref[0]
    comb = jnp.concatenate([tsi, avg], axis=1)
    r = jax.nn.sigmoid(jnp.dot(comb, Wr_WT_ref[...],
                               preferred_element_type=jnp.float32) + Wr_b_ref[...])
    z = jax.nn.sigmoid(jnp.dot(comb, Wz_WT_ref[...],
                               preferred_element_type=jnp.float32) + Wz_b_ref[...])
    comb2 = jnp.concatenate([tsi, avg * r], axis=1)
    ht = jnp.tanh(jnp.dot(comb2, Wt_WT_ref[...],
                          preferred_element_type=jnp.float32) + Wt_b_ref[...])
    hn = z * ht + (1.0 - z) * tsi
    rows = jax.lax.broadcasted_iota(jnp.int32, (64, 64), 0)
    cols = jax.lax.broadcasted_iota(jnp.int32, (64, 64), 1)
    eye = (rows == cols).astype(jnp.float32)
    hcol = jax.lax.dot_general(eye, hn, (((1,), (1,)), ((), ())),
                               preferred_element_type=jnp.float32)
    out_ref[0] = jnp.broadcast_to(hcol, out_ref.shape[1:])


@functools.partial(jax.jit, static_argnames=())
def kernel(tem_emb, init_hidden, mlp_W, mlp_b, Wr_W, Wr_b, Wz_W, Wz_b,
           Wt_W, Wt_b, year):
    b, n, d = tem_emb.shape
    k = _TOPK
    hn2 = n // 2
    x2 = tem_emb.reshape(b, hn2, 2 * d)
    hrow = init_hidden.reshape(1, d)
    h2 = jnp.concatenate([hrow, hrow], axis=1)

    feats = pl.pallas_call(
        _topk_kernel,
        grid=(b,),
        in_specs=[
            pl.BlockSpec((1, hn2, 2 * d), lambda i: (i, 0, 0)),
            pl.BlockSpec((1, 2 * d), lambda i: (0, 0)),
        ],
        out_specs=pl.BlockSpec((1, k, d), lambda i: (i, 0, 0)),
        out_shape=jax.ShapeDtypeStruct((b, k, d), jnp.float32),
    )(x2, h2)

    flat = jnp.transpose(feats, (0, 2, 1)).reshape(b, 1, d * k)
    avg = jnp.broadcast_to(hrow[None], (b, 1, d))

    out = pl.pallas_call(
        _head_kernel,
        grid=(b,),
        in_specs=[
            pl.BlockSpec((1, 1, d * k), lambda i: (i, 0, 0)),
            pl.BlockSpec((d * k, 64), lambda i: (0, 0)),
            pl.BlockSpec((1, 64), lambda i: (0, 0)),
            pl.BlockSpec((1, 1, d), lambda i: (i, 0, 0)),
            pl.BlockSpec((2 * 64, 64), lambda i: (0, 0)),
            pl.BlockSpec((1, 64), lambda i: (0, 0)),
            pl.BlockSpec((2 * 64, 64), lambda i: (0, 0)),
            pl.BlockSpec((1, 64), lambda i: (0, 0)),
            pl.BlockSpec((2 * 64, 64), lambda i: (0, 0)),
            pl.BlockSpec((1, 64), lambda i: (0, 0)),
        ],
        out_specs=pl.BlockSpec((1, 64, n), lambda i: (i, 0, 0)),
        out_shape=jax.ShapeDtypeStruct((b, 64, n), jnp.float32),
    )(flat, mlp_W.T, mlp_b.reshape(1, 64), avg,
      Wr_W.T, Wr_b.reshape(1, 64), Wz_W.T, Wz_b.reshape(1, 64),
      Wt_W.T, Wt_b.reshape(1, 64))
    return out
